# baseline (speedup 1.0000x reference)
import functools

import jax
import jax.numpy as jnp
from jax import lax
from jax.experimental import pallas as pl
from jax.experimental.pallas import tpu as pltpu
from jax.experimental.pallas import tpu_sc as plsc

VOCAB = 1000
D = 4
DK = 8
TPAD = 56
VEC = 16
WCHUNK = 896


def _make_gather(n_tokens: int):
    info = plsc.get_sparse_core_info()
    nw = info.num_cores * info.num_subcores
    assert n_tokens % (nw * 2 * WCHUNK) == 0
    b_per_w = n_tokens // nw
    n_pair = b_per_w // (2 * WCHUNK)
    nvec = WCHUNK // VEC
    mesh = plsc.VectorSubcoreMesh(core_axis_name="c", subcore_axis_name="s")

    @functools.partial(
        pl.kernel,
        mesh=mesh,
        out_type=jax.ShapeDtypeStruct((DK, n_tokens), jnp.float32),
        compiler_params=pltpu.CompilerParams(use_tc_tiling_on_sc=False, needs_layout_passes=False),
        scratch_types=(
            [pltpu.VMEM((VOCAB * DK,), jnp.float32)]
            + [pltpu.VMEM((b_per_w,), jnp.int32)]
            + [pltpu.VMEM((DK, WCHUNK), jnp.float32) for _ in range(2)]
            + [pltpu.SemaphoreType.DMA] * 2
        ),
    )
    def gather_kernel(ids_hbm, emb_hbm, out_hbm, emb_v, ids_v, wb0, wb1,
                      l_sem, w_sem):
        wid = lax.axis_index("s") * info.num_cores + lax.axis_index("c")
        base = wid * b_per_w
        pltpu.async_copy(emb_hbm, emb_v, l_sem)
        pltpu.async_copy(ids_hbm.at[pl.ds(base, b_per_w)], ids_v, l_sem)
        pltpu.make_async_copy(emb_hbm, emb_v, l_sem).wait()
        pltpu.make_async_copy(ids_hbm.at[pl.ds(0, b_per_w)], ids_v,
                              l_sem).wait()

        def fill(cc, wb):
            def vec_body(v, carry):
                ids16 = ids_v[pl.ds(cc * WCHUNK + v * VEC, VEC)]
                flat = ids16 * DK
                for d in range(DK):
                    vals = plsc.load_gather(emb_v, [flat + d])
                    wb[d, pl.ds(v * VEC, VEC)] = vals
                return carry

            lax.fori_loop(0, nvec, vec_body, 0)

        def flush(cc, wb):
            pltpu.async_copy(
                wb, out_hbm.at[:, pl.ds(base + cc * WCHUNK, WCHUNK)], w_sem)

        def wb_drain(n):
            for _ in range(n):
                pltpu.make_async_copy(
                    wb0, out_hbm.at[:, pl.ds(0, WCHUNK)], w_sem).wait()

        def body(p, carry):
            @pl.when(p >= 1)
            def _():
                wb_drain(2)

            fill(2 * p, wb0)
            flush(2 * p, wb0)
            fill(2 * p + 1, wb1)
            flush(2 * p + 1, wb1)
            return carry

        lax.fori_loop(0, n_pair, body, 0)
        wb_drain(2)

    return gather_kernel


def _matmul_body(h_ref, w_ref, out_ref):
    blk = h_ref.shape[1]
    mm = lax.dot_general(
        w_ref[...], h_ref[...],
        dimension_numbers=(((1,), (0,)), ((), ())),
        preferred_element_type=jnp.float32,
    )
    out_ref[...] = mm.reshape(1, VOCAB, blk)


def _pass_body(h_ref, w_ref, y_ref, out_ref):
    _matmul_body(h_ref, w_ref, out_ref)


def _projection_a(ht, head8, b, t, nt):
    blk = 2048
    nblk = b // blk
    return pl.pallas_call(
        _matmul_body,
        grid=(nt, nblk),
        in_specs=[
            pl.BlockSpec((DK, blk), lambda ti, bi: (0, ti * nblk + bi)),
            pl.BlockSpec((VOCAB, DK), lambda ti, bi: (0, 0)),
        ],
        out_specs=pl.BlockSpec((1, VOCAB, blk), lambda ti, bi: (ti, 0, bi)),
        out_shape=jax.ShapeDtypeStruct((t, VOCAB, b), jnp.float32),
    )(ht, head8)


def _projection_b(ht, head8, y, b, t, t0):
    blk = 2048
    nblk = b // blk
    return pl.pallas_call(
        _pass_body,
        grid=(t - t0, nblk),
        in_specs=[
            pl.BlockSpec((DK, blk), lambda ti, bi: (0, ti * nblk + bi)),
            pl.BlockSpec((VOCAB, DK), lambda ti, bi: (0, 0)),
            pl.BlockSpec(memory_space=pl.ANY),
        ],
        out_specs=pl.BlockSpec((1, VOCAB, blk),
                               lambda ti, bi: (ti + t0, 0, bi)),
        out_shape=jax.ShapeDtypeStruct((t, VOCAB, b), jnp.float32),
        input_output_aliases={2: 0},
    )(ht, head8, y)


def kernel(input_ids, emb_weight, head_weight):
    b, t = input_ids.shape
    ids_pad = jnp.pad(input_ids.astype(jnp.int32), ((0, 0), (0, TPAD - t)))
    ids_flat = ids_pad.T.reshape(-1)
    emb8 = jnp.pad(emb_weight, ((0, 0), (0, DK - D))).reshape(-1)
    head8 = jnp.pad(head_weight, ((0, 0), (0, DK - D)))
    tsplit = TPAD // 2
    na = tsplit * b
    gather = _make_gather(na)
    ht_a = gather(ids_flat[:na], emb8)
    ht_b = gather(ids_flat[na:], emb8)
    y = _projection_a(ht_a, head8, b, t, tsplit)
    y = _projection_b(ht_b, head8, y, b, t, tsplit)
    return jnp.transpose(y, (2, 0, 1))


# speedup vs baseline: 1.0017x; 1.0017x over previous
import functools

import jax
import jax.numpy as jnp
from jax import lax
from jax.experimental import pallas as pl
from jax.experimental.pallas import tpu as pltpu
from jax.experimental.pallas import tpu_sc as plsc

VOCAB = 1000
D = 4
DK = 8
TPAD = 56
VEC = 16
WCHUNK = 896


def _make_gather(n_tokens: int):
    info = plsc.get_sparse_core_info()
    nw = info.num_cores * info.num_subcores
    assert n_tokens % (nw * 2 * WCHUNK) == 0
    b_per_w = n_tokens // nw
    n_pair = b_per_w // (2 * WCHUNK)
    nvec = WCHUNK // VEC
    mesh = plsc.VectorSubcoreMesh(core_axis_name="c", subcore_axis_name="s")

    @functools.partial(
        pl.kernel,
        mesh=mesh,
        out_type=jax.ShapeDtypeStruct((DK, n_tokens), jnp.float32),
        compiler_params=pltpu.CompilerParams(use_tc_tiling_on_sc=False, needs_layout_passes=False),
        scratch_types=(
            [pltpu.VMEM((VOCAB * DK,), jnp.float32)]
            + [pltpu.VMEM((b_per_w,), jnp.int32)]
            + [pltpu.VMEM((DK, WCHUNK), jnp.float32) for _ in range(2)]
            + [pltpu.SemaphoreType.DMA] * 2
        ),
    )
    def gather_kernel(ids_hbm, emb_hbm, out_hbm, emb_v, ids_v, wb0, wb1,
                      l_sem, w_sem):
        wid = lax.axis_index("s") * info.num_cores + lax.axis_index("c")
        base = wid * b_per_w
        pltpu.async_copy(emb_hbm, emb_v, l_sem)
        pltpu.async_copy(ids_hbm.at[pl.ds(base, b_per_w)], ids_v, l_sem)
        pltpu.make_async_copy(emb_hbm, emb_v, l_sem).wait()
        pltpu.make_async_copy(ids_hbm.at[pl.ds(0, b_per_w)], ids_v,
                              l_sem).wait()

        def fill(cc, wb):
            def vec_body(v, carry):
                ids16 = ids_v[pl.ds(cc * WCHUNK + v * VEC, VEC)]
                flat = ids16 * DK
                for d in range(DK):
                    vals = plsc.load_gather(emb_v, [flat + d])
                    wb[d, pl.ds(v * VEC, VEC)] = vals
                return carry

            lax.fori_loop(0, nvec, vec_body, 0)

        def flush(cc, wb):
            pltpu.async_copy(
                wb, out_hbm.at[:, pl.ds(base + cc * WCHUNK, WCHUNK)], w_sem)

        def wb_drain(n):
            for _ in range(n):
                pltpu.make_async_copy(
                    wb0, out_hbm.at[:, pl.ds(0, WCHUNK)], w_sem).wait()

        def body(p, carry):
            @pl.when(p >= 1)
            def _():
                wb_drain(2)

            fill(2 * p, wb0)
            flush(2 * p, wb0)
            fill(2 * p + 1, wb1)
            flush(2 * p + 1, wb1)
            return carry

        lax.fori_loop(0, n_pair, body, 0)
        wb_drain(2)

    return gather_kernel


def _matmul_body(h_ref, w_ref, out_ref):
    blk = h_ref.shape[1]
    mm = lax.dot_general(
        w_ref[...], h_ref[...],
        dimension_numbers=(((1,), (0,)), ((), ())),
        preferred_element_type=jnp.float32,
    )
    out_ref[...] = mm.reshape(1, VOCAB, blk)


def _pass_body(h_ref, w_ref, y_ref, out_ref):
    _matmul_body(h_ref, w_ref, out_ref)


def _projection_a(ht, head8, b, t, nt):
    blk = 2048
    nblk = b // blk
    return pl.pallas_call(
        _matmul_body,
        grid=(nt, nblk),
        in_specs=[
            pl.BlockSpec((DK, blk), lambda ti, bi: (0, ti * nblk + bi)),
            pl.BlockSpec((VOCAB, DK), lambda ti, bi: (0, 0)),
        ],
        out_specs=pl.BlockSpec((1, VOCAB, blk), lambda ti, bi: (ti, 0, bi)),
        out_shape=jax.ShapeDtypeStruct((t, VOCAB, b), jnp.float32),
    )(ht, head8)


def _projection_b(ht, head8, y, b, t, t0):
    blk = 2048
    nblk = b // blk
    return pl.pallas_call(
        _pass_body,
        grid=(t - t0, nblk),
        in_specs=[
            pl.BlockSpec((DK, blk), lambda ti, bi: (0, ti * nblk + bi)),
            pl.BlockSpec((VOCAB, DK), lambda ti, bi: (0, 0)),
            pl.BlockSpec(memory_space=pl.ANY),
        ],
        out_specs=pl.BlockSpec((1, VOCAB, blk),
                               lambda ti, bi: (ti + t0, 0, bi)),
        out_shape=jax.ShapeDtypeStruct((t, VOCAB, b), jnp.float32),
        input_output_aliases={2: 0},
    )(ht, head8, y)


def kernel(input_ids, emb_weight, head_weight):
    b, t = input_ids.shape
    ids_pad = jnp.pad(input_ids.astype(jnp.int32), ((0, 0), (0, TPAD - t)))
    ids_flat = ids_pad.T.reshape(-1)
    emb8 = jnp.pad(emb_weight, ((0, 0), (0, DK - D))).reshape(-1)
    head8 = jnp.pad(head_weight, ((0, 0), (0, DK - D)))
    tsplit = TPAD // 2
    na = tsplit * b
    gather = _make_gather(na)
    ht_a = gather(ids_flat[:na], emb8)
    y = _projection_a(ht_a, head8, b, t, tsplit)
    ht_b = gather(ids_flat[na:], emb8)
    y = _projection_b(ht_b, head8, y, b, t, tsplit)
    return jnp.transpose(y, (2, 0, 1))


# speedup vs baseline: 1.0329x; 1.0312x over previous
import functools

import jax
import jax.numpy as jnp
from jax import lax
from jax.experimental import pallas as pl
from jax.experimental.pallas import tpu as pltpu
from jax.experimental.pallas import tpu_sc as plsc

VOCAB = 1000
D = 4
DK = 8
TPAD = 56
VEC = 16
WCHUNK = 800


def _make_gather(n_tokens: int):
    info = plsc.get_sparse_core_info()
    nw = info.num_cores * info.num_subcores
    assert n_tokens % (nw * 2 * WCHUNK) == 0
    b_per_w = n_tokens // nw
    n_pair = b_per_w // (2 * WCHUNK)
    nvec = WCHUNK // VEC
    mesh = plsc.VectorSubcoreMesh(core_axis_name="c", subcore_axis_name="s")

    @functools.partial(
        pl.kernel,
        mesh=mesh,
        out_type=jax.ShapeDtypeStruct((DK, n_tokens), jnp.float32),
        compiler_params=pltpu.CompilerParams(use_tc_tiling_on_sc=False, needs_layout_passes=False),
        scratch_types=(
            [pltpu.VMEM((VOCAB * DK,), jnp.float32)]
            + [pltpu.VMEM((b_per_w,), jnp.int32)]
            + [pltpu.VMEM((DK, WCHUNK), jnp.float32) for _ in range(2)]
            + [pltpu.SemaphoreType.DMA] * 2
        ),
    )
    def gather_kernel(ids_hbm, emb_hbm, out_hbm, emb_v, ids_v, wb0, wb1,
                      l_sem, w_sem):
        wid = lax.axis_index("s") * info.num_cores + lax.axis_index("c")
        base = wid * b_per_w
        pltpu.async_copy(emb_hbm, emb_v, l_sem)
        pltpu.async_copy(ids_hbm.at[pl.ds(base, b_per_w)], ids_v, l_sem)
        pltpu.make_async_copy(emb_hbm, emb_v, l_sem).wait()
        pltpu.make_async_copy(ids_hbm.at[pl.ds(0, b_per_w)], ids_v,
                              l_sem).wait()

        def fill(cc, wb):
            def vec_body(v, carry):
                ids16 = ids_v[pl.ds(cc * WCHUNK + v * VEC, VEC)]
                flat = ids16 * DK
                for d in range(DK):
                    vals = plsc.load_gather(emb_v, [flat + d])
                    wb[d, pl.ds(v * VEC, VEC)] = vals
                return carry

            lax.fori_loop(0, nvec, vec_body, 0)

        def flush(cc, wb):
            pltpu.async_copy(
                wb, out_hbm.at[:, pl.ds(base + cc * WCHUNK, WCHUNK)], w_sem)

        def wb_drain(n):
            for _ in range(n):
                pltpu.make_async_copy(
                    wb0, out_hbm.at[:, pl.ds(0, WCHUNK)], w_sem).wait()

        def body(p, carry):
            @pl.when(p >= 1)
            def _():
                wb_drain(2)

            fill(2 * p, wb0)
            flush(2 * p, wb0)
            fill(2 * p + 1, wb1)
            flush(2 * p + 1, wb1)
            return carry

        lax.fori_loop(0, n_pair, body, 0)
        wb_drain(2)

    return gather_kernel


def _matmul_body(h_ref, w_ref, out_ref):
    blk = h_ref.shape[1]
    mm = lax.dot_general(
        w_ref[...], h_ref[...],
        dimension_numbers=(((1,), (0,)), ((), ())),
        preferred_element_type=jnp.float32,
    )
    out_ref[...] = mm.reshape(1, VOCAB, blk)


def _projection(ht, head8, b, t):
    blk = 2048
    nblk = b // blk
    return pl.pallas_call(
        _matmul_body,
        grid=(t, nblk),
        in_specs=[
            pl.BlockSpec((DK, blk), lambda ti, bi: (0, ti * nblk + bi)),
            pl.BlockSpec((VOCAB, DK), lambda ti, bi: (0, 0)),
        ],
        out_specs=pl.BlockSpec((1, VOCAB, blk), lambda ti, bi: (ti, 0, bi)),
        out_shape=jax.ShapeDtypeStruct((t, VOCAB, b), jnp.float32),
    )(ht, head8)


def kernel(input_ids, emb_weight, head_weight):
    b, t = input_ids.shape
    ids_flat = input_ids.astype(jnp.int32).T.reshape(-1)
    emb8 = jnp.pad(emb_weight, ((0, 0), (0, DK - D))).reshape(-1)
    head8 = jnp.pad(head_weight, ((0, 0), (0, DK - D)))
    ht = _make_gather(b * t)(ids_flat, emb8)
    y = _projection(ht, head8, b, t)
    return jnp.transpose(y, (2, 0, 1))


# speedup vs baseline: 1.0341x; 1.0012x over previous
import functools

import jax
import jax.numpy as jnp
from jax import lax
from jax.experimental import pallas as pl
from jax.experimental.pallas import tpu as pltpu
from jax.experimental.pallas import tpu_sc as plsc

VOCAB = 1000
D = 4
DK = 8
VEC = 16
WCHUNK = 800


def _make_gather(n_tokens: int):
    info = plsc.get_sparse_core_info()
    nw = info.num_cores * info.num_subcores
    assert n_tokens % (nw * 2 * WCHUNK) == 0
    b_per_w = n_tokens // nw
    n_pair = b_per_w // (2 * WCHUNK)
    nvec = WCHUNK // VEC
    mesh = plsc.VectorSubcoreMesh(core_axis_name="c", subcore_axis_name="s")

    @functools.partial(
        pl.kernel,
        mesh=mesh,
        out_type=jax.ShapeDtypeStruct((DK, n_tokens), jnp.float32),
        compiler_params=pltpu.CompilerParams(use_tc_tiling_on_sc=False, needs_layout_passes=False),
        scratch_types=(
            [pltpu.VMEM((VOCAB * DK,), jnp.float32)]
            + [pltpu.VMEM((b_per_w,), jnp.int32)]
            + [pltpu.VMEM((DK, WCHUNK), jnp.float32) for _ in range(2)]
            + [pltpu.SemaphoreType.DMA] * 2
        ),
    )
    def gather_kernel(ids_hbm, emb_hbm, out_hbm, emb_v, ids_v, wb0, wb1,
                      l_sem, w_sem):
        wid = lax.axis_index("s") * info.num_cores + lax.axis_index("c")
        base = wid * b_per_w
        pltpu.async_copy(emb_hbm, emb_v, l_sem)
        pltpu.async_copy(ids_hbm.at[pl.ds(base, b_per_w)], ids_v, l_sem)
        pltpu.make_async_copy(emb_hbm, emb_v, l_sem).wait()
        pltpu.make_async_copy(ids_hbm.at[pl.ds(0, b_per_w)], ids_v,
                              l_sem).wait()

        def fill(cc, wb):
            def vec_body(v, carry):
                ids16 = ids_v[pl.ds(cc * WCHUNK + v * VEC, VEC)]
                flat = ids16 * DK
                for d in range(DK):
                    vals = plsc.load_gather(emb_v, [flat + d])
                    wb[d, pl.ds(v * VEC, VEC)] = vals
                return carry

            lax.fori_loop(0, nvec, vec_body, 0)

        def flush(cc, wb):
            pltpu.async_copy(
                wb, out_hbm.at[:, pl.ds(base + cc * WCHUNK, WCHUNK)], w_sem)

        def wb_drain(n):
            for _ in range(n):
                pltpu.make_async_copy(
                    wb0, out_hbm.at[:, pl.ds(0, WCHUNK)], w_sem).wait()

        def body(p, carry):
            @pl.when(p >= 1)
            def _():
                wb_drain(2)

            fill(2 * p, wb0)
            flush(2 * p, wb0)
            fill(2 * p + 1, wb1)
            flush(2 * p + 1, wb1)
            return carry

        lax.fori_loop(0, n_pair, body, 0)
        wb_drain(2)

    return gather_kernel


def _matmul_body(h_ref, w_ref, out_ref):
    blk = h_ref.shape[1]
    mm = lax.dot_general(
        w_ref[...], h_ref[...],
        dimension_numbers=(((1,), (0,)), ((), ())),
        preferred_element_type=jnp.float32,
    )
    out_ref[...] = mm.reshape(1, VOCAB, blk)


def _projection(ht, head8, b, t):
    blk = 2048
    nblk = b // blk
    return pl.pallas_call(
        _matmul_body,
        grid=(t, nblk),
        in_specs=[
            pl.BlockSpec((DK, blk), lambda ti, bi: (0, ti * nblk + bi)),
            pl.BlockSpec((VOCAB, DK), lambda ti, bi: (0, 0)),
        ],
        out_specs=pl.BlockSpec((1, VOCAB, blk), lambda ti, bi: (ti, 0, bi)),
        out_shape=jax.ShapeDtypeStruct((t, VOCAB, b), jnp.float32),
    )(ht, head8)


def kernel(input_ids, emb_weight, head_weight):
    b, t = input_ids.shape
    ids_flat = input_ids.astype(jnp.int32).T.reshape(-1)
    emb8 = jnp.pad(emb_weight, ((0, 0), (0, DK - D))).reshape(-1)
    head8 = jnp.pad(head_weight, ((0, 0), (0, DK - D)))
    ht = _make_gather(b * t)(ids_flat, emb8)
    y = _projection(ht, head8, b, t)
    return jnp.transpose(y, (2, 0, 1))
